# Initial kernel scaffold; baseline (speedup 1.0000x reference)
import jax
import jax.numpy as jnp
from jax.experimental import pallas as pl


def kernel(query, pos_items, pop_prob, table, num_neg):
    raise NotImplementedError("write your pallas kernel here")


# speedup vs baseline: 291.1939x; 291.1939x over previous
import functools

import numpy as np
import jax
import jax.numpy as jnp
from jax import lax
from jax.experimental import pallas as pl
from jax.experimental.pallas import tpu as pltpu
from jax.experimental.pallas import tpu_sc as plsc

NC = 2
NS = 16
NW = NC * NS
L = 16


def _log_body(x_ref, o_ref):
    o_ref[...] = jnp.log(x_ref[...])


def _tc_log(x2d, blk_rows):
    rows, cols = x2d.shape
    return pl.pallas_call(
        _log_body,
        out_shape=jax.ShapeDtypeStruct((rows, cols), jnp.float32),
        grid=(rows // blk_rows,),
        in_specs=[pl.BlockSpec((blk_rows, cols), lambda i: (i, 0))],
        out_specs=pl.BlockSpec((blk_rows, cols), lambda i: (i, 0)),
    )(x2d)


@functools.lru_cache(maxsize=None)
def _build_sc_sampler(nseed, nq, rows, cpow, nitems, B):
    nblk = nseed // NW // B
    groups = B // L
    chunks = B // 128
    posb = nq // NW
    pos_groups = posb // L
    pos_chunks = posb // 128
    steps = int(np.log2(cpow))

    mesh = plsc.VectorSubcoreMesh(
        core_axis_name="c", subcore_axis_name="s",
        num_cores=NC, num_subcores=NS)

    @functools.partial(
        pl.kernel,
        out_type=(
            jax.ShapeDtypeStruct((nseed,), jnp.int32),
            jax.ShapeDtypeStruct((nseed,), jnp.float32),
            jax.ShapeDtypeStruct((nq,), jnp.float32),
        ),
        mesh=mesh,
        compiler_params=pltpu.CompilerParams(
            needs_layout_passes=False, use_tc_tiling_on_sc=False),
        scratch_types=[
            pltpu.VMEM((cpow,), jnp.float32),
            pltpu.VMEM((B,), jnp.float32),
            pltpu.VMEM((B,), jnp.int32),
            pltpu.VMEM((B,), jnp.int32),
            pltpu.VMEM((B,), jnp.int32),
            pltpu.VMEM((B, L), jnp.float32),
            pltpu.VMEM((B,), jnp.int32),
            pltpu.VMEM((B,), jnp.float32),
            pltpu.VMEM((L,), jnp.int32),
            pltpu.SemaphoreType.DMA,
        ],
    )
    def sampler(seeds_hbm, positems_hbm, coarse_hbm, t2_hbm, popf_hbm,
                offv_hbm, items_out, pvals_out, pospv_out,
                coarse_v, seeds_v, pos_v, fidx_v, sel_v, f2_v, oi_v, op_v,
                off_v, sem):
        wid = lax.axis_index("s") * NC + lax.axis_index("c")
        pltpu.sync_copy(coarse_hbm, coarse_v)
        pltpu.sync_copy(offv_hbm, off_v)
        iota = lax.iota(jnp.int32, L)

        def coarse_search(s):
            pos = jnp.zeros((L,), jnp.int32)
            for k in range(steps - 1, -1, -1):
                step = 1 << k
                v = plsc.load_gather(coarse_v, [pos + (step - 1)])
                pos = pos + jnp.where(v < s, step, 0)
            return pos

        pbase = pl.multiple_of(wid * posb, 8)
        pltpu.sync_copy(positems_hbm.at[pl.ds(pbase, posb)],
                        pos_v.at[pl.ds(0, posb)])

        def pos_sel_body(g, carry):
            goff = pl.multiple_of(g * L, L)
            it = pos_v[pl.ds(goff, L)]
            sel = jnp.clip(jnp.where(it >= nitems, -1, it) + 1, 0, nitems)
            sel_v[pl.ds(goff, L)] = sel
            return carry
        lax.fori_loop(0, pos_groups, pos_sel_body, 0)

        pdescs = [
            pltpu.async_copy(popf_hbm.at[sel_v.at[pl.ds(c * 128, 128)]],
                             op_v.at[pl.ds(c * 128, 128)], sem)
            for c in range(pos_chunks)]
        for d in pdescs:
            d.wait()
        pltpu.sync_copy(op_v.at[pl.ds(0, posb)],
                        pospv_out.at[pl.ds(pbase, posb)])

        def blk_body(b, carry):
            sbase = pl.multiple_of(wid * (nblk * B) + b * B, 8)
            pltpu.sync_copy(seeds_hbm.at[pl.ds(sbase, B)], seeds_v)

            def p_coarse(g, c2):
                goff = pl.multiple_of(g * L, L)
                s = seeds_v[pl.ds(goff, L)]
                pos = coarse_search(s)
                pos_v[pl.ds(goff, L)] = pos
                fidx_v[pl.ds(goff, L)] = jnp.maximum(pos - 1, 0)
                return c2
            lax.fori_loop(0, groups, p_coarse, 0)

            ds1 = [
                pltpu.async_copy(t2_hbm.at[fidx_v.at[pl.ds(c * 128, 128)]],
                                 f2_v.at[pl.ds(c * 128, 128)], sem)
                for c in range(chunks)]
            for d in ds1:
                d.wait()

            def p_fine(g, c2):
                goff = pl.multiple_of(g * L, L)
                pos = pos_v[pl.ds(goff, L)]
                cnt = jnp.zeros((L,), jnp.int32)
                for l in range(L):
                    w = f2_v[goff + l, :]
                    sl = plsc.load_gather(seeds_v, [
                        jnp.full((L,), goff + l, jnp.int32)])
                    c = plsc.all_reduce_population_count(w < sl)
                    cnt = cnt + jnp.where(iota == l, c, 0)
                ans = jnp.maximum(16 * pos - 15, 0) + cnt
                item = ans - 1 + off_v[...]
                oi_v[pl.ds(goff, L)] = item
                sel_v[pl.ds(goff, L)] = jnp.clip(
                    jnp.where(item >= nitems, -1, item) + 1, 0, nitems)
                return c2
            lax.fori_loop(0, groups, p_fine, 0)

            ds2 = [
                pltpu.async_copy(popf_hbm.at[sel_v.at[pl.ds(c * 128, 128)]],
                                 op_v.at[pl.ds(c * 128, 128)], sem)
                for c in range(chunks)]
            for d in ds2:
                d.wait()

            pltpu.sync_copy(oi_v, items_out.at[pl.ds(sbase, B)])
            pltpu.sync_copy(op_v, pvals_out.at[pl.ds(sbase, B)])
            return carry
        lax.fori_loop(0, nblk, blk_body, 0)

    return sampler


def kernel(query, pos_items, pop_prob, table, num_neg):
    nq = int(np.prod(query.shape[:-1]))
    nneg_static = 200
    nitems = pop_prob.shape[0] - 1
    tbl = table.shape[0]
    rows = (tbl + L - 1) // L
    cpow = 1 << int(np.ceil(np.log2(rows + 1)))
    nseed = nq * nneg_static

    seeds = jax.random.uniform(
        jax.random.key(42), (nq, nneg_static), dtype=jnp.float32)
    seeds_flat = seeds.reshape(-1)

    inf = jnp.full((1,), jnp.inf, jnp.float32)
    coarse = jnp.concatenate(
        [table[::L], jnp.broadcast_to(inf, (cpow - rows,))])
    t2 = jnp.concatenate(
        [table[1:], jnp.broadcast_to(inf, (rows * L - (tbl - 1),))]
    ).reshape(rows, L)
    popf = jnp.concatenate(
        [pop_prob, jnp.ones((rows * L - tbl,), jnp.float32)])
    offv = jnp.full((L,), jnp.asarray(num_neg, jnp.int32) - nneg_static,
                    jnp.int32)

    sampler = _build_sc_sampler(nseed, nq, rows, cpow, nitems, 2048)
    items, pvals, pospv = sampler(
        seeds_flat, pos_items.astype(jnp.int32), coarse, t2, popf, offv)

    neg_items = items.reshape(query.shape[:-1] + (nneg_static,))
    neg_prob = _tc_log(pvals.reshape(-1, 1024), 128).reshape(
        query.shape[:-1] + (nneg_static,))
    pos_prob = _tc_log(pospv.reshape(-1, 1024), min(nq // 1024, 128)
                       ).reshape(query.shape[:-1])
    return (pos_prob, neg_items, neg_prob)
